# baseline (speedup 1.0000x reference)
import functools

import jax
import jax.numpy as jnp
from jax import lax
from jax.experimental import layout as jex_layout
from jax.experimental import pallas as pl
from jax.experimental.pallas import tpu as pltpu
from jax.experimental.pallas import tpu_sc as plsc

VOCAB = 1000000
D = 32
H = 128
T = 9
B = 4096
L = 50

NC = 2
NS = 16
NW = NC * NS
BL = B * L
PER_W = BL // NW
CHUNK = 128
NCH = PER_W // CHUNK

_REPACK_ROWS = 4000


def _repack_body(src_ref, dst_ref):
    q = _REPACK_ROWS // 4
    dst_ref[...] = jnp.concatenate(
        [src_ref[k * q:(k + 1) * q, :] for k in range(4)], axis=1
    )


def _repack_table(emb_table):
    return pl.pallas_call(
        _repack_body,
        grid=(VOCAB // _REPACK_ROWS,),
        in_specs=[pl.BlockSpec((_REPACK_ROWS, D), lambda i: (i, 0))],
        out_specs=pl.BlockSpec((_REPACK_ROWS // 4, 4 * D), lambda i: (i, 0)),
        out_shape=jax.ShapeDtypeStruct((VOCAB // 4, 4 * D), jnp.float32),
        compiler_params=pltpu.CompilerParams(
            dimension_semantics=("arbitrary",),
        ),
    )(emb_table)


@functools.cache
def _sc_gather_fn():
    mesh = plsc.VectorSubcoreMesh(core_axis_name="c", subcore_axis_name="s")
    return pl.kernel(
        _sc_gather_body,
        mesh=mesh,
        out_type=jax.ShapeDtypeStruct((BL, 4 * D), jnp.float32),
        scratch_types=[
            pltpu.VMEM((NCH, CHUNK), jnp.int32),
            pltpu.VMEM((2, CHUNK, D), jnp.float32),
            pltpu.SemaphoreType.DMA,
            pltpu.SemaphoreType.DMA,
        ],
        compiler_params=pltpu.CompilerParams(use_tc_tiling_on_sc=False),
    )


def _sc_gather_body(table_hbm, idx_hbm, out_hbm, idx_v, rows_v, sem0, sem1):
    wid = lax.axis_index("s") * NC + lax.axis_index("c")
    base = wid * PER_W
    pltpu.sync_copy(idx_hbm.at[wid], idx_v)

    pltpu.async_copy(table_hbm.at[idx_v.at[0]], rows_v.at[0], sem0)

    def body(jj, _):
        j0 = jj * 2
        pltpu.async_copy(table_hbm.at[idx_v.at[j0 + 1]], rows_v.at[1], sem1)
        pltpu.make_async_copy(
            table_hbm.at[idx_v.at[j0]], rows_v.at[0], sem0
        ).wait()
        pltpu.sync_copy(
            rows_v.at[0],
            out_hbm.at[pl.ds(base + j0 * CHUNK, CHUNK), pl.ds(0, D)],
        )

        @pl.when(j0 + 2 < NCH)
        def _():
            pltpu.async_copy(table_hbm.at[idx_v.at[j0 + 2]], rows_v.at[0], sem0)

        pltpu.make_async_copy(
            table_hbm.at[idx_v.at[j0 + 1]], rows_v.at[1], sem1
        ).wait()
        pltpu.sync_copy(
            rows_v.at[1],
            out_hbm.at[pl.ds(base + (j0 + 1) * CHUNK, CHUNK), pl.ds(0, D)],
        )
        return 0

    lax.fori_loop(0, NCH // 2, body, 0)


def _lstm_body(x_ref, wih_ref, whh_ref, b_ref, wfc_ref, bfc_ref, out_ref,
               h_scr, c_scr):
    t = pl.program_id(0)

    @pl.when(t == 0)
    def _():
        h_scr[...] = jnp.zeros_like(h_scr)
        c_scr[...] = jnp.zeros_like(c_scr)

    x_t = x_ref[0][:, :D]
    h = h_scr[...]
    gates = (
        jnp.dot(x_t, wih_ref[...], preferred_element_type=jnp.float32)
        + jnp.dot(h, whh_ref[...], preferred_element_type=jnp.float32)
        + b_ref[...]
    )
    i = jax.nn.sigmoid(gates[:, :H])
    f = jax.nn.sigmoid(gates[:, H:2 * H])
    g = jnp.tanh(gates[:, 2 * H:3 * H])
    o = jax.nn.sigmoid(gates[:, 3 * H:])
    c = f * c_scr[...] + i * g
    h = o * jnp.tanh(c)
    h_scr[...] = h
    c_scr[...] = c

    logits = jnp.dot(h, wfc_ref[...], preferred_element_type=jnp.float32) + bfc_ref[...]
    m = jnp.max(logits, axis=-1, keepdims=True)
    lse = jnp.log(jnp.sum(jnp.exp(logits - m), axis=-1, keepdims=True)) + m
    out_ref[0] = logits - lse


def _lstm_fc(x, wih_t, whh_t, b, wfc_t, bfc):
    return pl.pallas_call(
        _lstm_body,
        grid=(L,),
        in_specs=[
            pl.BlockSpec((1, B, 4 * D), lambda t: (t, 0, 0)),
            pl.BlockSpec((D, 4 * H), lambda t: (0, 0)),
            pl.BlockSpec((H, 4 * H), lambda t: (0, 0)),
            pl.BlockSpec((1, 4 * H), lambda t: (0, 0)),
            pl.BlockSpec((H, T), lambda t: (0, 0)),
            pl.BlockSpec((1, T), lambda t: (0, 0)),
        ],
        out_specs=pl.BlockSpec((1, B, T), lambda t: (t, 0, 0)),
        out_shape=jax.ShapeDtypeStruct((L, B, T), jnp.float32),
        scratch_shapes=[
            pltpu.VMEM((B, H), jnp.float32),
            pltpu.VMEM((B, H), jnp.float32),
        ],
        compiler_params=pltpu.CompilerParams(
            dimension_semantics=("arbitrary",),
        ),
    )(x, wih_t, whh_t, b, wfc_t, bfc)


def kernel(sentences, labels, emb_table, W_ih, W_hh, b_ih, b_hh, W_fc, b_fc):
    del labels
    emb_table = jex_layout.with_layout_constraint(
        emb_table,
        jex_layout.Layout(major_to_minor=(1, 0), tiling=((8, 128),)),
    )
    idx = jnp.swapaxes(sentences, 0, 1).astype(jnp.int32)
    q = _REPACK_ROWS // 4
    rem = idx % _REPACK_ROWS
    idx = (idx // _REPACK_ROWS * q + idx % q) * 4 + rem // q
    idx = idx.reshape(NW, NCH, CHUNK)
    emb_lin = _repack_table(emb_table).reshape(VOCAB, D)
    x_pad = _sc_gather_fn()(emb_lin, idx)
    x = x_pad.reshape(L, B, 4 * D)

    wih_t = W_ih.T
    whh_t = W_hh.T
    b = (b_ih + b_hh).reshape(1, 4 * H)
    wfc_t = W_fc.T
    bfc = b_fc.reshape(1, T)

    out_lbt = _lstm_fc(x, wih_t, whh_t, b, wfc_t, bfc)
    return jnp.swapaxes(out_lbt, 0, 1)


# speedup vs baseline: 1.5020x; 1.5020x over previous
import functools

import jax
import jax.numpy as jnp
from jax import lax
from jax.experimental import layout as jex_layout
from jax.experimental import pallas as pl
from jax.experimental.pallas import tpu as pltpu
from jax.experimental.pallas import tpu_sc as plsc

VOCAB = 1000000
D = 32
H = 128
T = 9
B = 4096
L = 50

NC = 2
NS = 16
NW = NC * NS
BL = B * L
PER_W = BL // NW
CHUNK = 128
NCH = PER_W // CHUNK

_REPACK_ROWS = 4096
_REPACK_GRID = -(-VOCAB // _REPACK_ROWS)
VOCAB_PAD = _REPACK_GRID * _REPACK_ROWS


def _repack_body(src_ref, dst_ref):
    q = _REPACK_ROWS // 4
    st = jnp.swapaxes(src_ref[...], 0, 1)
    dst_ref[...] = jnp.concatenate(
        [st[k * q:(k + 1) * q, :] for k in range(4)], axis=1
    )


def _repack_table(emb_table_t):
    return pl.pallas_call(
        _repack_body,
        grid=(_REPACK_GRID,),
        in_specs=[pl.BlockSpec((D, _REPACK_ROWS), lambda i: (0, i))],
        out_specs=pl.BlockSpec((_REPACK_ROWS // 4, 4 * D), lambda i: (i, 0)),
        out_shape=jax.ShapeDtypeStruct((VOCAB_PAD // 4, 4 * D), jnp.float32),
        compiler_params=pltpu.CompilerParams(
            dimension_semantics=("arbitrary",),
        ),
    )(emb_table_t)


@functools.cache
def _sc_gather_fn():
    mesh = plsc.VectorSubcoreMesh(core_axis_name="c", subcore_axis_name="s")
    return pl.kernel(
        _sc_gather_body,
        mesh=mesh,
        out_type=jax.ShapeDtypeStruct((BL, 4 * D), jnp.float32),
        scratch_types=[
            pltpu.VMEM((NCH, CHUNK), jnp.int32),
            pltpu.VMEM((2, CHUNK, D), jnp.float32),
            pltpu.SemaphoreType.DMA,
            pltpu.SemaphoreType.DMA,
        ],
        compiler_params=pltpu.CompilerParams(use_tc_tiling_on_sc=False),
    )


def _sc_gather_body(table_hbm, idx_hbm, out_hbm, idx_v, rows_v, sem0, sem1):
    wid = lax.axis_index("s") * NC + lax.axis_index("c")
    base = wid * PER_W
    pltpu.sync_copy(idx_hbm.at[wid], idx_v)

    pltpu.async_copy(table_hbm.at[idx_v.at[0]], rows_v.at[0], sem0)

    def body(jj, _):
        j0 = jj * 2
        pltpu.async_copy(table_hbm.at[idx_v.at[j0 + 1]], rows_v.at[1], sem1)
        pltpu.make_async_copy(
            table_hbm.at[idx_v.at[j0]], rows_v.at[0], sem0
        ).wait()
        pltpu.sync_copy(
            rows_v.at[0],
            out_hbm.at[pl.ds(base + j0 * CHUNK, CHUNK), pl.ds(0, D)],
        )

        @pl.when(j0 + 2 < NCH)
        def _():
            pltpu.async_copy(table_hbm.at[idx_v.at[j0 + 2]], rows_v.at[0], sem0)

        pltpu.make_async_copy(
            table_hbm.at[idx_v.at[j0 + 1]], rows_v.at[1], sem1
        ).wait()
        pltpu.sync_copy(
            rows_v.at[1],
            out_hbm.at[pl.ds(base + (j0 + 1) * CHUNK, CHUNK), pl.ds(0, D)],
        )
        return 0

    lax.fori_loop(0, NCH // 2, body, 0)


def _lstm_body(x_ref, wih_ref, whh_ref, b_ref, wfc_ref, bfc_ref, out_ref,
               h_scr, c_scr):
    t = pl.program_id(0)

    @pl.when(t == 0)
    def _():
        h_scr[...] = jnp.zeros_like(h_scr)
        c_scr[...] = jnp.zeros_like(c_scr)

    x_t = x_ref[0][:, :D]
    h = h_scr[...]
    gates = (
        jnp.dot(x_t, wih_ref[...], preferred_element_type=jnp.float32)
        + jnp.dot(h, whh_ref[...], preferred_element_type=jnp.float32)
        + b_ref[...]
    )
    i = jax.nn.sigmoid(gates[:, :H])
    f = jax.nn.sigmoid(gates[:, H:2 * H])
    g = jnp.tanh(gates[:, 2 * H:3 * H])
    o = jax.nn.sigmoid(gates[:, 3 * H:])
    c = f * c_scr[...] + i * g
    h = o * jnp.tanh(c)
    h_scr[...] = h
    c_scr[...] = c

    logits = jnp.dot(h, wfc_ref[...], preferred_element_type=jnp.float32) + bfc_ref[...]
    m = jnp.max(logits, axis=-1, keepdims=True)
    lse = jnp.log(jnp.sum(jnp.exp(logits - m), axis=-1, keepdims=True)) + m
    out_ref[0] = logits - lse


def _lstm_fc(x, wih_t, whh_t, b, wfc_t, bfc):
    return pl.pallas_call(
        _lstm_body,
        grid=(L,),
        in_specs=[
            pl.BlockSpec((1, B, 4 * D), lambda t: (t, 0, 0)),
            pl.BlockSpec((D, 4 * H), lambda t: (0, 0)),
            pl.BlockSpec((H, 4 * H), lambda t: (0, 0)),
            pl.BlockSpec((1, 4 * H), lambda t: (0, 0)),
            pl.BlockSpec((H, T), lambda t: (0, 0)),
            pl.BlockSpec((1, T), lambda t: (0, 0)),
        ],
        out_specs=pl.BlockSpec((1, B, T), lambda t: (t, 0, 0)),
        out_shape=jax.ShapeDtypeStruct((L, B, T), jnp.float32),
        scratch_shapes=[
            pltpu.VMEM((B, H), jnp.float32),
            pltpu.VMEM((B, H), jnp.float32),
        ],
        compiler_params=pltpu.CompilerParams(
            dimension_semantics=("arbitrary",),
        ),
    )(x, wih_t, whh_t, b, wfc_t, bfc)


def kernel(sentences, labels, emb_table, W_ih, W_hh, b_ih, b_hh, W_fc, b_fc):
    del labels
    idx = jnp.swapaxes(sentences, 0, 1).astype(jnp.int32)
    q = _REPACK_ROWS // 4
    rem = idx % _REPACK_ROWS
    idx = (idx // _REPACK_ROWS * q + idx % q) * 4 + rem // q
    idx = idx.reshape(NW, NCH, CHUNK)
    emb_lin = _repack_table(jnp.swapaxes(emb_table, 0, 1)).reshape(VOCAB_PAD, D)
    x_pad = _sc_gather_fn()(emb_lin, idx)
    x = x_pad.reshape(L, B, 4 * D)

    wih_t = W_ih.T
    whh_t = W_hh.T
    b = (b_ih + b_hh).reshape(1, 4 * H)
    wfc_t = W_fc.T
    bfc = b_fc.reshape(1, T)

    out_lbt = _lstm_fc(x, wih_t, whh_t, b, wfc_t, bfc)
    return jnp.swapaxes(out_lbt, 0, 1)


# speedup vs baseline: 1.6221x; 1.0799x over previous
import functools

import jax
import jax.numpy as jnp
from jax import lax
from jax.experimental import layout as jex_layout
from jax.experimental import pallas as pl
from jax.experimental.pallas import tpu as pltpu
from jax.experimental.pallas import tpu_sc as plsc

VOCAB = 1000000
D = 32
H = 128
T = 9
B = 4096
L = 50

NC = 2
NS = 16
NW = NC * NS
BL = B * L
PER_W = BL // NW
CHUNK = 128
NCH = PER_W // CHUNK

_REPACK_ROWS = 4096
_REPACK_GRID = -(-VOCAB // _REPACK_ROWS)
VOCAB_PAD = _REPACK_GRID * _REPACK_ROWS


def _repack_body(src_ref, eye_ref, dst_ref):
    q = _REPACK_ROWS // 4
    acc = jnp.zeros((q, 4 * D), jnp.float32)
    for k in range(4):
        acc = acc + jax.lax.dot_general(
            src_ref[:, k * q:(k + 1) * q], eye_ref[k],
            (((0,), (0,)), ((), ())),
            preferred_element_type=jnp.float32,
        )
    dst_ref[...] = acc


def _repack_table(emb_table_t):
    eye = (jnp.arange(4 * D)[None, None, :]
           == jnp.arange(4)[:, None, None] * D
           + jnp.arange(D)[None, :, None]).astype(jnp.float32)
    return pl.pallas_call(
        _repack_body,
        grid=(_REPACK_GRID,),
        in_specs=[
            pl.BlockSpec((D, _REPACK_ROWS), lambda i: (0, i)),
            pl.BlockSpec((4, D, 4 * D), lambda i: (0, 0, 0)),
        ],
        out_specs=pl.BlockSpec((_REPACK_ROWS // 4, 4 * D), lambda i: (i, 0)),
        out_shape=jax.ShapeDtypeStruct((VOCAB_PAD // 4, 4 * D), jnp.float32),
        compiler_params=pltpu.CompilerParams(
            dimension_semantics=("arbitrary",),
        ),
    )(emb_table_t, eye)


@functools.cache
def _sc_gather_fn():
    mesh = plsc.VectorSubcoreMesh(core_axis_name="c", subcore_axis_name="s")
    return pl.kernel(
        _sc_gather_body,
        mesh=mesh,
        out_type=jax.ShapeDtypeStruct((BL, 4 * D), jnp.float32),
        scratch_types=[
            pltpu.VMEM((NCH, CHUNK), jnp.int32),
            pltpu.VMEM((2, CHUNK, D), jnp.float32),
            pltpu.SemaphoreType.DMA,
            pltpu.SemaphoreType.DMA,
        ],
        compiler_params=pltpu.CompilerParams(use_tc_tiling_on_sc=False),
    )


def _sc_gather_body(table_hbm, idx_hbm, out_hbm, idx_v, rows_v, sem0, sem1):
    wid = lax.axis_index("s") * NC + lax.axis_index("c")
    base = wid * PER_W
    pltpu.sync_copy(idx_hbm.at[wid], idx_v)

    pltpu.async_copy(table_hbm.at[idx_v.at[0]], rows_v.at[0], sem0)

    def body(jj, _):
        j0 = jj * 2
        pltpu.async_copy(table_hbm.at[idx_v.at[j0 + 1]], rows_v.at[1], sem1)
        pltpu.make_async_copy(
            table_hbm.at[idx_v.at[j0]], rows_v.at[0], sem0
        ).wait()
        pltpu.sync_copy(
            rows_v.at[0],
            out_hbm.at[pl.ds(base + j0 * CHUNK, CHUNK), pl.ds(0, D)],
        )

        @pl.when(j0 + 2 < NCH)
        def _():
            pltpu.async_copy(table_hbm.at[idx_v.at[j0 + 2]], rows_v.at[0], sem0)

        pltpu.make_async_copy(
            table_hbm.at[idx_v.at[j0 + 1]], rows_v.at[1], sem1
        ).wait()
        pltpu.sync_copy(
            rows_v.at[1],
            out_hbm.at[pl.ds(base + (j0 + 1) * CHUNK, CHUNK), pl.ds(0, D)],
        )
        return 0

    lax.fori_loop(0, NCH // 2, body, 0)


def _lstm_body(x_ref, wih_ref, whh_ref, b_ref, wfc_ref, bfc_ref, out_ref,
               h_scr, c_scr):
    t = pl.program_id(0)

    @pl.when(t == 0)
    def _():
        h_scr[...] = jnp.zeros_like(h_scr)
        c_scr[...] = jnp.zeros_like(c_scr)

    x_t = x_ref[0][:, :D]
    h = h_scr[...]
    gates = (
        jnp.dot(x_t, wih_ref[...], preferred_element_type=jnp.float32)
        + jnp.dot(h, whh_ref[...], preferred_element_type=jnp.float32)
        + b_ref[...]
    )

    def sig(v):
        return 0.5 * jnp.tanh(0.5 * v) + 0.5

    i = sig(gates[:, :H])
    f = sig(gates[:, H:2 * H])
    g = jnp.tanh(gates[:, 2 * H:3 * H])
    o = sig(gates[:, 3 * H:])
    c = f * c_scr[...] + i * g
    h = o * jnp.tanh(c)
    h_scr[...] = h
    c_scr[...] = c

    logits = jnp.dot(h, wfc_ref[...], preferred_element_type=jnp.float32) + bfc_ref[...]
    m = jnp.max(logits, axis=-1, keepdims=True)
    lse = jnp.log(jnp.sum(jnp.exp(logits - m), axis=-1, keepdims=True)) + m
    out_ref[0] = logits - lse


def _lstm_fc(x, wih_t, whh_t, b, wfc_t, bfc):
    return pl.pallas_call(
        _lstm_body,
        grid=(L,),
        in_specs=[
            pl.BlockSpec((1, B, 4 * D), lambda t: (t, 0, 0)),
            pl.BlockSpec((D, 4 * H), lambda t: (0, 0)),
            pl.BlockSpec((H, 4 * H), lambda t: (0, 0)),
            pl.BlockSpec((1, 4 * H), lambda t: (0, 0)),
            pl.BlockSpec((H, T), lambda t: (0, 0)),
            pl.BlockSpec((1, T), lambda t: (0, 0)),
        ],
        out_specs=pl.BlockSpec((1, B, T), lambda t: (t, 0, 0)),
        out_shape=jax.ShapeDtypeStruct((L, B, T), jnp.float32),
        scratch_shapes=[
            pltpu.VMEM((B, H), jnp.float32),
            pltpu.VMEM((B, H), jnp.float32),
        ],
        compiler_params=pltpu.CompilerParams(
            dimension_semantics=("arbitrary",),
        ),
    )(x, wih_t, whh_t, b, wfc_t, bfc)


def kernel(sentences, labels, emb_table, W_ih, W_hh, b_ih, b_hh, W_fc, b_fc):
    del labels
    idx = jnp.swapaxes(sentences, 0, 1).astype(jnp.int32)
    q = _REPACK_ROWS // 4
    rem = idx % _REPACK_ROWS
    idx = (idx // _REPACK_ROWS * q + idx % q) * 4 + rem // q
    idx = idx.reshape(NW, NCH, CHUNK)
    emb_lin = _repack_table(jnp.swapaxes(emb_table, 0, 1)).reshape(VOCAB_PAD, D)
    x_pad = _sc_gather_fn()(emb_lin, idx)
    x = x_pad.reshape(L, B, 4 * D)

    wih_t = W_ih.T
    whh_t = W_hh.T
    b = (b_ih + b_hh).reshape(1, 4 * H)
    wfc_t = W_fc.T
    bfc = b_fc.reshape(1, T)

    out_lbt = _lstm_fc(x, wih_t, whh_t, b, wfc_t, bfc)
    return jnp.swapaxes(out_lbt, 0, 1)
